# Initial kernel scaffold; baseline (speedup 1.0000x reference)
import jax
import jax.numpy as jnp
from jax.experimental import pallas as pl


def kernel(node_features, edge_attributes, lstm_W_ih, lstm_W_hh, lstm_b_ih, lstm_b_hh, gcn_W, gcn_b, gat_W, gat_att_src, gat_att_dst, gat_b, edge_index):
    raise NotImplementedError("write your pallas kernel here")


# speedup vs baseline: 1.0023x; 1.0023x over previous
import functools

import jax
import jax.numpy as jnp
from jax.experimental import pallas as pl
from jax.experimental.pallas import tpu as pltpu

N = 10000
T = 128
H = 32
C = 32
HEADS = 3

NPAD = 10240


def _lstm_body(x_ref, wih_ref, whh_ref, b_ref, out_ref, h_scr, c_scr):
    t = pl.program_id(0)

    @pl.when(t == 0)
    def _init():
        h_scr[:, :] = jnp.zeros_like(h_scr)
        c_scr[:, :] = jnp.zeros_like(c_scr)

    h = h_scr[:, :]
    c = c_scr[:, :]
    g = jnp.dot(whh_ref[:, :], h, preferred_element_type=jnp.float32)
    g = g + wih_ref[:, :] * x_ref[0, :, :] + b_ref[:, :]
    i = jax.nn.sigmoid(g[0:H, :])
    f = jax.nn.sigmoid(g[H:2 * H, :])
    gg = jnp.tanh(g[2 * H:3 * H, :])
    o = jax.nn.sigmoid(g[3 * H:4 * H, :])
    c = f * c + i * gg
    h = o * jnp.tanh(c)
    h_scr[:, :] = h
    c_scr[:, :] = c

    @pl.when(t == T - 1)
    def _emit():
        out_ref[:, :] = h


def _lstm_last_hidden(node_features, lstm_W_ih, lstm_W_hh, lstm_b_ih, lstm_b_hh):
    n = node_features.shape[0]
    xt = jnp.transpose(node_features)
    xt = jnp.pad(xt, ((0, 0), (0, NPAD - n))).reshape(T, 1, NPAD)
    b = (lstm_b_ih + lstm_b_hh)[:, None]

    out = pl.pallas_call(
        _lstm_body,
        grid=(T,),
        in_specs=[
            pl.BlockSpec((1, 1, NPAD), lambda t: (t, 0, 0)),
            pl.BlockSpec((4 * H, 1), lambda t: (0, 0)),
            pl.BlockSpec((4 * H, H), lambda t: (0, 0)),
            pl.BlockSpec((4 * H, 1), lambda t: (0, 0)),
        ],
        out_specs=pl.BlockSpec((H, NPAD), lambda t: (0, 0)),
        out_shape=jax.ShapeDtypeStruct((H, NPAD), jnp.float32),
        scratch_shapes=[
            pltpu.VMEM((H, NPAD), jnp.float32),
            pltpu.VMEM((H, NPAD), jnp.float32),
        ],
    )(xt, lstm_W_ih, lstm_W_hh, b)
    return jnp.transpose(out)[:n]


def kernel(node_features, edge_attributes, lstm_W_ih, lstm_W_hh, lstm_b_ih,
           lstm_b_hh, gcn_W, gcn_b, gat_W, gat_att_src, gat_att_dst, gat_b,
           edge_index):
    n = node_features.shape[0]
    h = _lstm_last_hidden(node_features, lstm_W_ih, lstm_W_hh, lstm_b_ih,
                          lstm_b_hh)
    x = jax.nn.relu(h)

    loops = jnp.arange(n)
    src = jnp.concatenate([edge_index[0], loops])
    dst = jnp.concatenate([edge_index[1], loops])

    ew = jnp.concatenate([edge_attributes, jnp.ones((n,), x.dtype)])
    deg = jax.ops.segment_sum(ew, dst, num_segments=n)
    dinv = deg ** -0.5
    norm = dinv[src] * ew * dinv[dst]
    xw = x @ gcn_W
    agg = jax.ops.segment_sum(xw[src] * norm[:, None], dst, num_segments=n)
    x = jax.nn.relu(agg + gcn_b)

    xh = (x @ gat_W).reshape(n, HEADS, C)
    a_src = (xh * gat_att_src[None]).sum(-1)
    a_dst = (xh * gat_att_dst[None]).sum(-1)
    alpha = a_src[src] + a_dst[dst]
    alpha = jnp.where(alpha > 0, alpha, 0.2 * alpha)
    amax = jax.ops.segment_max(alpha, dst, num_segments=n)
    ex = jnp.exp(alpha - amax[dst])
    den = jax.ops.segment_sum(ex, dst, num_segments=n)
    attn = ex / (den[dst] + 1e-16)
    out = jax.ops.segment_sum(xh[src] * attn[:, :, None], dst, num_segments=n)
    out = out.reshape(n, HEADS * C) + gat_b
    return out.mean(axis=0, keepdims=True)


# speedup vs baseline: 10.2758x; 10.2518x over previous
import functools

import jax
import jax.numpy as jnp
from jax import lax
from jax.experimental import pallas as pl
from jax.experimental.pallas import tpu as pltpu
from jax.experimental.pallas import tpu_sc as plsc

N = 10000
T = 128
H = 32
C = 32
HEADS = 3
E = 320000

NPAD = 10240

NW = 32
CHUNK = 80
KDEPTH = 5


def _lstm_body(x_ref, wih_ref, whh_ref, b_ref, out_ref, h_scr, c_scr):
    t = pl.program_id(0)

    @pl.when(t == 0)
    def _init():
        h_scr[:, :] = jnp.zeros_like(h_scr)
        c_scr[:, :] = jnp.zeros_like(c_scr)

    h = h_scr[:, :]
    c = c_scr[:, :]
    g = jnp.dot(whh_ref[:, :], h, preferred_element_type=jnp.float32)
    g = g + wih_ref[:, :] * x_ref[0, :, :] + b_ref[:, :]
    i = jax.nn.sigmoid(g[0:H, :])
    f = jax.nn.sigmoid(g[H:2 * H, :])
    gg = jnp.tanh(g[2 * H:3 * H, :])
    o = jax.nn.sigmoid(g[3 * H:4 * H, :])
    c = f * c + i * gg
    h = o * jnp.tanh(c)
    h_scr[:, :] = h
    c_scr[:, :] = c

    @pl.when(t == T - 1)
    def _emit():
        out_ref[:, :] = h


def _lstm_last_hidden(node_features, lstm_W_ih, lstm_W_hh, lstm_b_ih, lstm_b_hh):
    n = node_features.shape[0]
    xt = jnp.transpose(node_features)
    xt = jnp.pad(xt, ((0, 0), (0, NPAD - n))).reshape(T, 1, NPAD)
    b = (lstm_b_ih + lstm_b_hh)[:, None]

    out = pl.pallas_call(
        _lstm_body,
        grid=(T,),
        in_specs=[
            pl.BlockSpec((1, 1, NPAD), lambda t: (t, 0, 0)),
            pl.BlockSpec((4 * H, 1), lambda t: (0, 0)),
            pl.BlockSpec((4 * H, H), lambda t: (0, 0)),
            pl.BlockSpec((4 * H, 1), lambda t: (0, 0)),
        ],
        out_specs=pl.BlockSpec((H, NPAD), lambda t: (0, 0)),
        out_shape=jax.ShapeDtypeStruct((H, NPAD), jnp.float32),
        scratch_shapes=[
            pltpu.VMEM((H, NPAD), jnp.float32),
            pltpu.VMEM((H, NPAD), jnp.float32),
        ],
    )(xt, lstm_W_ih, lstm_W_hh, b)
    return jnp.transpose(out)[:n]


def _make_sc_gather(D, n_edges):
    per_w = n_edges // NW
    n_ch = per_w // CHUNK
    n_blk = n_ch // KDEPTH
    assert per_w * NW == n_edges and CHUNK * n_ch == per_w
    assert KDEPTH * n_blk == n_ch
    mesh = plsc.VectorSubcoreMesh(core_axis_name="c", subcore_axis_name="s")

    @functools.partial(
        pl.kernel, mesh=mesh,
        out_type=jax.ShapeDtypeStruct((n_edges, D), jnp.float32),
        scratch_types=[
            pltpu.VMEM((per_w,), jnp.int32),
            pltpu.VMEM((KDEPTH, CHUNK, D), jnp.float32),
            pltpu.SemaphoreType.DMA,
            pltpu.SemaphoreType.DMA,
        ],
    )
    def gather_k(table_hbm, idx_hbm, out_hbm, idx_v, buf_v, gsem, wsem):
        wid = lax.axis_index("s") * 2 + lax.axis_index("c")
        base = wid * per_w
        pltpu.sync_copy(idx_hbm.at[pl.ds(base, per_w)], idx_v)

        def block(bi, carry):
            e0 = bi * (KDEPTH * CHUNK)
            for j in range(KDEPTH):
                pltpu.async_copy(
                    table_hbm.at[idx_v.at[pl.ds(e0 + j * CHUNK, CHUNK)]],
                    buf_v.at[j], gsem)
            for j in range(KDEPTH):
                pltpu.make_async_copy(
                    table_hbm.at[idx_v.at[pl.ds(e0 + j * CHUNK, CHUNK)]],
                    buf_v.at[j], gsem).wait()
            for j in range(KDEPTH):
                pltpu.async_copy(
                    buf_v.at[j],
                    out_hbm.at[pl.ds(base + e0 + j * CHUNK, CHUNK)], wsem)
            for j in range(KDEPTH):
                pltpu.make_async_copy(
                    buf_v.at[j],
                    out_hbm.at[pl.ds(base + e0 + j * CHUNK, CHUNK)], wsem).wait()
            return carry

        lax.fori_loop(0, n_blk, block, 0)

    return gather_k


_GATHERS = {}


def _sc_gather(table, idx, D):
    key = (D, idx.shape[0])
    if key not in _GATHERS:
        _GATHERS[key] = _make_sc_gather(D, idx.shape[0])
    return _GATHERS[key](table, idx)


def _leaky(v):
    return jnp.where(v > 0, v, 0.2 * v)


def kernel(node_features, edge_attributes, lstm_W_ih, lstm_W_hh, lstm_b_ih,
           lstm_b_hh, gcn_W, gcn_b, gat_W, gat_att_src, gat_att_dst, gat_b,
           edge_index):
    n = node_features.shape[0]
    src = edge_index[0]
    dst = edge_index[1]
    ew = edge_attributes

    h = _lstm_last_hidden(node_features, lstm_W_ih, lstm_W_hh, lstm_b_ih,
                          lstm_b_hh)
    x = jax.nn.relu(h)

    deg = jax.ops.segment_sum(ew, dst, num_segments=n) + 1.0
    dinv = deg ** -0.5
    xwn = (x @ gcn_W) * dinv[:, None]
    xwn_tab = jnp.pad(xwn, ((0, 0), (0, 128 - H)))
    g1 = _sc_gather(xwn_tab, src, 128)[:, :H]
    agg = jax.ops.segment_sum(g1 * ew[:, None], dst, num_segments=n)
    x2 = jax.nn.relu(dinv[:, None] * (agg + xwn) + gcn_b)

    xh = x2 @ gat_W
    xh3 = xh.reshape(n, HEADS, C)
    a_src = (xh3 * gat_att_src[None]).sum(-1)
    a_dst = (xh3 * gat_att_dst[None]).sum(-1)
    amax_global = a_src.max(axis=0)
    m = _leaky(amax_global[None, :] + a_dst)

    src_tab = jnp.concatenate(
        [xh, a_src, jnp.zeros((n, 29), jnp.float32)], axis=1)
    ga = _sc_gather(src_tab, src, 128)
    xh_src = ga[:, :96]
    asrc_e = ga[:, 96:99]

    dst_tab = jnp.concatenate(
        [a_dst, m, jnp.zeros((n, 122), jnp.float32)], axis=1)
    gd = _sc_gather(dst_tab, dst, 128)
    adst_e = gd[:, :3]
    m_e = gd[:, 3:6]

    ex_e = jnp.exp(_leaky(asrc_e + adst_e) - m_e)
    ex_self = jnp.exp(_leaky(a_src + a_dst) - m)
    den = jax.ops.segment_sum(ex_e, dst, num_segments=n) + ex_self

    den_tab = jnp.concatenate(
        [den, jnp.zeros((n, 125), jnp.float32)], axis=1)
    gden = _sc_gather(den_tab, dst, 128)
    attn_e = ex_e / (gden[:, :3] + 1e-16)

    edge_part = (xh_src.reshape(-1, HEADS, C) * attn_e[:, :, None]).sum(axis=0)
    attn_self = ex_self / (den + 1e-16)
    self_part = (xh3 * attn_self[:, :, None]).sum(axis=0)
    out = (edge_part + self_part).reshape(1, HEADS * C) / n + gat_b[None, :]
    return out
